# baseline (speedup 1.0000x reference)
import functools

import jax
import jax.numpy as jnp
from jax import lax
from jax.experimental import pallas as pl
from jax.experimental.pallas import tpu as pltpu
from jax.experimental.pallas import tpu_sc as plsc

_N_HEADS = 8
_TABLE_SIZE = 100000
_D_EMBED = 32

_info = plsc.get_sparse_core_info()
_NC, _NS, _L = _info.num_cores, _info.num_subcores, _info.num_lanes
_NW = _NC * _NS

_CHUNK = 1024


def _transpose_table(wt):
    n, bl = wt.shape[1], 3200

    def body(x_ref, o_ref):
        t3 = x_ref[...].T.reshape(bl // 4, 4, _D_EMBED)
        o_ref[...] = jnp.concatenate([t3[:, c, :] for c in range(4)], axis=1)

    return pl.pallas_call(
        body,
        grid=(n // bl,),
        in_specs=[pl.BlockSpec((_D_EMBED, bl), lambda i: (0, i))],
        out_specs=pl.BlockSpec((bl // 4, 128), lambda i: (i, 0)),
        out_shape=jax.ShapeDtypeStruct((n // 4, 128), jnp.float32),
    )(wt)


def _sc_gather(T, n_blk_b):
    mesh = plsc.VectorSubcoreMesh(core_axis_name="c", subcore_axis_name="s")
    B = n_blk_b * 128

    @functools.partial(
        pl.kernel,
        mesh=mesh,
        compiler_params=pltpu.CompilerParams(use_tc_tiling_on_sc=False),
        out_type=jax.ShapeDtypeStruct((T, _N_HEADS, B, _D_EMBED), jnp.float32),
        scratch_types=[
            pltpu.VMEM((_CHUNK,), jnp.int32),
            pltpu.VMEM((_CHUNK,), jnp.int32),
            pltpu.VMEM((_CHUNK, _D_EMBED), jnp.float32),
            pltpu.VMEM((_CHUNK, _D_EMBED), jnp.float32),
            pltpu.SemaphoreType.DMA,
            pltpu.SemaphoreType.DMA,
            pltpu.SemaphoreType.DMA,
            pltpu.SemaphoreType.DMA,
            pltpu.SemaphoreType.DMA,
            pltpu.SemaphoreType.DMA,
        ],
    )
    def k(idx_hbm, w_hbm, out_hbm, idx_a, idx_b, rows_a, rows_b,
          si_a, si_b, sg_a, sg_b, ss_a, ss_b):
        wid = lax.axis_index("s") * _NC + lax.axis_index("c")

        def add_offsets(idx_v):
            for h in range(_N_HEADS):
                off = jnp.full((_L,), h * _TABLE_SIZE, jnp.int32)
                for u in range(128 // _L):
                    s = pl.ds(h * 128 + u * _L, _L)
                    idx_v[s] = idx_v[s] + off

        def start_idx(t, idx_v, sem):
            pltpu.async_copy(idx_hbm.at[t, wid], idx_v, sem)

        def wait_idx(idx_v, sem):
            pltpu.make_async_copy(idx_hbm.at[0, 0], idx_v, sem).wait()

        def start_gather(idx_v, rows_v, sem):
            pltpu.async_copy(w_hbm.at[idx_v], rows_v, sem)

        def wait_gather(rows_v, sem):
            pltpu.make_async_copy(w_hbm.at[pl.ds(0, _CHUNK)], rows_v,
                                  sem).wait()

        def start_store(t, rows_v, sem):
            for h in range(_N_HEADS):
                pltpu.async_copy(
                    rows_v.at[pl.ds(h * 128, 128)],
                    out_hbm.at[t, h, pl.ds(wid * 128, 128)], sem)

        def wait_store(rows_v, sem):
            pltpu.make_async_copy(rows_v, out_hbm.at[0, 0, pl.ds(0, _CHUNK)],
                                  sem).wait()

        start_idx(0, idx_a, si_a)
        wait_idx(idx_a, si_a)
        add_offsets(idx_a)
        start_gather(idx_a, rows_a, sg_a)
        start_idx(1, idx_b, si_b)

        def pair_body(i, carry):
            t = i * 2

            wait_idx(idx_b, si_b)
            add_offsets(idx_b)

            @pl.when(t > 0)
            def _():
                wait_store(rows_b, ss_b)
            wait_gather(rows_a, sg_a)
            start_store(t, rows_a, ss_a)
            start_gather(idx_b, rows_b, sg_b)

            @pl.when(t + 2 < T)
            def _():
                start_idx(t + 2, idx_a, si_a)

            @pl.when(t + 2 < T)
            def _():
                wait_idx(idx_a, si_a)
                add_offsets(idx_a)
            wait_store(rows_a, ss_a)
            wait_gather(rows_b, sg_b)
            start_store(t + 1, rows_b, ss_b)

            @pl.when(t + 2 < T)
            def _():
                start_gather(idx_a, rows_a, sg_a)

            @pl.when(t + 3 < T)
            def _():
                start_idx(t + 3, idx_b, si_b)
            return carry

        lax.fori_loop(0, T // 2, pair_body, 0)
        wait_store(rows_b, ss_b)

    return k


def kernel(indices, weight):
    B, T, H = indices.shape
    n_blk_b = B // 128
    w_rm = _transpose_table(weight.T).reshape(weight.shape[0], _D_EMBED)
    idx3 = (indices.transpose(1, 2, 0)
            .reshape(T, H, n_blk_b, 128)
            .transpose(0, 2, 1, 3)
            .reshape(T, n_blk_b, H * 128)
            .astype(jnp.int32))
    outz = _sc_gather(T, n_blk_b)(idx3, w_rm)
    return outz.transpose(2, 0, 1, 3)


# speedup vs baseline: 1.1975x; 1.1975x over previous
import functools

import jax
import jax.numpy as jnp
from jax import lax
from jax.experimental import pallas as pl
from jax.experimental.pallas import tpu as pltpu
from jax.experimental.pallas import tpu_sc as plsc
from jax.experimental.layout import Layout, with_layout_constraint

_N_HEADS = 8
_TABLE_SIZE = 100000
_D_EMBED = 32

_info = plsc.get_sparse_core_info()
_NC, _NS, _L = _info.num_cores, _info.num_subcores, _info.num_lanes
_NW = _NC * _NS

_CHUNK = 1024


def _transpose_table(wt):
    n, bl = wt.shape[1], 3200

    def body(x_ref, o_ref):
        t3 = x_ref[...].T.reshape(bl // 4, 4, _D_EMBED)
        for c in range(4):
            o_ref[:, 32 * c:32 * (c + 1)] = t3[:, c, :]

    return pl.pallas_call(
        body,
        grid=(n // bl,),
        in_specs=[pl.BlockSpec((_D_EMBED, bl), lambda i: (0, i))],
        out_specs=pl.BlockSpec((bl // 4, 128), lambda i: (i, 0)),
        out_shape=jax.ShapeDtypeStruct((n // 4, 128), jnp.float32),
    )(wt)


def _sc_gather(T, n_blk_b):
    mesh = plsc.VectorSubcoreMesh(core_axis_name="c", subcore_axis_name="s")
    B = n_blk_b * 128

    @functools.partial(
        pl.kernel,
        mesh=mesh,
        compiler_params=pltpu.CompilerParams(use_tc_tiling_on_sc=False),
        out_type=jax.ShapeDtypeStruct((T, _N_HEADS, B, _D_EMBED), jnp.float32),
        scratch_types=[
            pltpu.VMEM((_CHUNK,), jnp.int32),
            pltpu.VMEM((_CHUNK,), jnp.int32),
            pltpu.VMEM((_CHUNK, _D_EMBED), jnp.float32),
            pltpu.VMEM((_CHUNK, _D_EMBED), jnp.float32),
            pltpu.SemaphoreType.DMA,
            pltpu.SemaphoreType.DMA,
            pltpu.SemaphoreType.DMA,
            pltpu.SemaphoreType.DMA,
            pltpu.SemaphoreType.DMA,
            pltpu.SemaphoreType.DMA,
        ],
    )
    def k(idx_hbm, w_hbm, out_hbm, idx_a, idx_b, rows_a, rows_b,
          si_a, si_b, sg_a, sg_b, ss_a, ss_b):
        wid = lax.axis_index("s") * _NC + lax.axis_index("c")

        def add_offsets(idx_v):
            for h in range(_N_HEADS):
                off = jnp.full((_L,), h * _TABLE_SIZE, jnp.int32)
                for u in range(128 // _L):
                    s = pl.ds(h * 128 + u * _L, _L)
                    idx_v[s] = idx_v[s] + off

        def start_idx(t, idx_v, sem):
            pltpu.async_copy(idx_hbm.at[t, wid], idx_v, sem)

        def wait_idx(idx_v, sem):
            pltpu.make_async_copy(idx_hbm.at[0, 0], idx_v, sem).wait()

        def start_gather(idx_v, rows_v, sem):
            pltpu.async_copy(w_hbm.at[idx_v], rows_v, sem)

        def wait_gather(rows_v, sem):
            pltpu.make_async_copy(w_hbm.at[pl.ds(0, _CHUNK)], rows_v,
                                  sem).wait()

        def start_store(t, rows_v, sem):
            for h in range(_N_HEADS):
                pltpu.async_copy(
                    rows_v.at[pl.ds(h * 128, 128)],
                    out_hbm.at[t, h, pl.ds(wid * 128, 128)], sem)

        def wait_store(rows_v, sem):
            pltpu.make_async_copy(rows_v, out_hbm.at[0, 0, pl.ds(0, _CHUNK)],
                                  sem).wait()

        start_idx(0, idx_a, si_a)
        wait_idx(idx_a, si_a)
        add_offsets(idx_a)
        start_gather(idx_a, rows_a, sg_a)
        start_idx(1, idx_b, si_b)

        def pair_body(i, carry):
            t = i * 2

            wait_idx(idx_b, si_b)
            add_offsets(idx_b)

            @pl.when(t > 0)
            def _():
                wait_store(rows_b, ss_b)
            wait_gather(rows_a, sg_a)
            start_store(t, rows_a, ss_a)
            start_gather(idx_b, rows_b, sg_b)

            @pl.when(t + 2 < T)
            def _():
                start_idx(t + 2, idx_a, si_a)

            @pl.when(t + 2 < T)
            def _():
                wait_idx(idx_a, si_a)
                add_offsets(idx_a)
            wait_store(rows_a, ss_a)
            wait_gather(rows_b, sg_b)
            start_store(t + 1, rows_b, ss_b)

            @pl.when(t + 2 < T)
            def _():
                start_gather(idx_a, rows_a, sg_a)

            @pl.when(t + 3 < T)
            def _():
                start_idx(t + 3, idx_b, si_b)
            return carry

        lax.fori_loop(0, T // 2, pair_body, 0)
        wait_store(rows_b, ss_b)

    return k


def kernel(indices, weight):
    B, T, H = indices.shape
    n_blk_b = B // 128
    w_rm = _transpose_table(weight.T).reshape(weight.shape[0], _D_EMBED)
    idx3 = (indices.transpose(1, 2, 0)
            .reshape(T, H, n_blk_b, 128)
            .transpose(0, 2, 1, 3)
            .reshape(T, n_blk_b, H * 128)
            .astype(jnp.int32))
    outz = _sc_gather(T, n_blk_b)(idx3, w_rm)
    out = outz.transpose(2, 0, 1, 3)
    return with_layout_constraint(
        out, Layout(major_to_minor=(1, 2, 0, 3)))
